# baseline (speedup 1.0000x reference)
import functools

import jax
import jax.numpy as jnp
from jax import lax
from jax.experimental import pallas as pl
from jax.experimental.pallas import tpu as pltpu
from jax.experimental.pallas import tpu_sc as plsc

N_BITS = 20
OUT_DIM = 16
BATCH = 16384
NUM_CORES = 2
NUM_SUBCORES = 16
NW = NUM_CORES * NUM_SUBCORES
B_PER_W = BATCH // NW
NBLK = B_PER_W // 128
LANES = 16
HALF = 8 * (2 ** N_BITS)


def _body(img_hbm, enc_hbm, out_hbm, img_v, scale_v, addr_v, gat_v,
          g0, g1, g2, g3, sem2):
    wid = lax.axis_index("s") * NUM_CORES + lax.axis_index("c")
    base_row = wid * B_PER_W
    gsems = [g0, g1, g2, g3]

    def img_fire(j, _):
        pltpu.async_copy(
            img_hbm.at[pl.ds(j * BATCH + base_row, B_PER_W)],
            img_v.at[pl.ds(j * B_PER_W, B_PER_W)], sem2)
        return _

    def img_wait(j, _):
        pltpu.make_async_copy(
            img_hbm.at[pl.ds(j * BATCH + base_row, B_PER_W)],
            img_v.at[pl.ds(j * B_PER_W, B_PER_W)], sem2).wait()
        return _

    lax.fori_loop(0, N_BITS, img_fire, 0)
    lax.fori_loop(0, N_BITS, img_wait, 0)

    for b2 in range(NBLK):
        def index_chunk(c8, _, b2=b2):
            cix = b2 * 8 + c8
            idx = jnp.zeros((LANES,), jnp.int32)
            acc = jnp.zeros((LANES,), jnp.float32)
            for j in range(N_BITS):
                g = img_v[pl.ds(j * B_PER_W + cix * LANES, LANES)]
                bit = jnp.full((LANES,), 1 << j, jnp.int32)
                idx = idx + jnp.where(g > 0, bit,
                                      jnp.zeros((LANES,), jnp.int32))
                acc = acc + jnp.abs(g)
            scale_v[pl.ds(cix * LANES, LANES)] = acc * (1.0 / N_BITS)
            ebase = ((idx >> 7) << 10) + (idx & 127)
            for h in range(2):
                for cc in range(8):
                    a = ebase + (h * HALF + cc * 128)
                    dst = (((h * NBLK + b2) * 8 + cc) * 8 + c8) * LANES
                    addr_v[pl.ds(dst, LANES)] = a
            return _

        def gat_fire(cc, _, b2=b2):
            for h in range(2):
                o = ((h * NBLK + b2) * 8 + cc) * 128
                pltpu.async_copy(enc_hbm.at[addr_v.at[pl.ds(o, 128)]],
                                 gat_v.at[pl.ds(o, 128)], gsems[b2])
            return _

        lax.fori_loop(0, 8, index_chunk, 0)
        lax.fori_loop(0, 8, gat_fire, 0)

    half_w = NBLK * 8 * 128
    for b2 in range(NBLK):
        def gat_wait(cc, _, b2=b2):
            for h in range(2):
                o = ((h * NBLK + b2) * 8 + cc) * 128
                pltpu.make_async_copy(enc_hbm.at[addr_v.at[pl.ds(o, 128)]],
                                      gat_v.at[pl.ds(o, 128)],
                                      gsems[b2]).wait()
            return _

        def scale_cc(cc, _, b2=b2):
            for h in range(2):
                for r8 in range(8):
                    o = ((h * NBLK + b2) * 8 + cc) * 128 + r8 * LANES
                    s = scale_v[pl.ds(b2 * 128 + r8 * LANES, LANES)]
                    gat_v[pl.ds(o, LANES)] = gat_v[pl.ds(o, LANES)] * s
            return _

        lax.fori_loop(0, 8, gat_wait, 0)
        lax.fori_loop(0, 8, scale_cc, 0)
        for h in range(2):
            src_o = (h * NBLK + b2) * 1024
            dst_o = h * (BATCH // 128) * 1024 + wid * half_w + b2 * 1024
            pltpu.async_copy(gat_v.at[pl.ds(src_o, 1024)],
                             out_hbm.at[pl.ds(dst_o, 1024)], sem2)
    for b2 in range(NBLK):
        for h in range(2):
            src_o = (h * NBLK + b2) * 1024
            dst_o = h * (BATCH // 128) * 1024 + wid * half_w + b2 * 1024
            pltpu.make_async_copy(gat_v.at[pl.ds(src_o, 1024)],
                                  out_hbm.at[pl.ds(dst_o, 1024)], sem2).wait()


@jax.jit
def kernel(image, encoding):
    mesh = plsc.VectorSubcoreMesh(
        core_axis_name="c", subcore_axis_name="s",
        num_cores=NUM_CORES, num_subcores=NUM_SUBCORES)
    k = functools.partial(
        pl.kernel,
        out_type=jax.ShapeDtypeStruct((2 * (BATCH // 128) * 1024,), jnp.float32),
        mesh=mesh,
        scratch_types=[
            pltpu.VMEM((B_PER_W * N_BITS,), jnp.float32),
            pltpu.VMEM((B_PER_W,), jnp.float32),
            pltpu.VMEM((B_PER_W * OUT_DIM,), jnp.int32),
            pltpu.VMEM((B_PER_W * OUT_DIM,), jnp.float32),
            pltpu.SemaphoreType.DMA,
            pltpu.SemaphoreType.DMA,
            pltpu.SemaphoreType.DMA,
            pltpu.SemaphoreType.DMA,
            pltpu.SemaphoreType.DMA,
        ],
        compiler_params=pltpu.CompilerParams(
            needs_layout_passes=False, use_tc_tiling_on_sc=False),
    )(_body)
    enc_flat = (encoding.reshape(8192, 128, 2, 8)
                .transpose(2, 0, 3, 1).reshape(-1))
    out = k(image.T.reshape(-1), enc_flat)
    return (out.reshape(2, BATCH // 128, 8, 128)
            .transpose(1, 3, 0, 2).reshape(BATCH, OUT_DIM))


# speedup vs baseline: 1.0837x; 1.0837x over previous
import functools

import jax
import jax.numpy as jnp
from jax import lax
from jax.experimental import pallas as pl
from jax.experimental.pallas import tpu as pltpu
from jax.experimental.pallas import tpu_sc as plsc

N_BITS = 20
OUT_DIM = 16
BATCH = 16384
NUM_CORES = 2
NUM_SUBCORES = 16
NW = NUM_CORES * NUM_SUBCORES
B_PER_W = BATCH // NW
NBLK = B_PER_W // 128
LANES = 16
HALF = 8 * (2 ** N_BITS)


def _body(img_hbm, enc_hbm, out_hbm, img_v, scale_v, addr_v, gat_v,
          sem2):
    wid = lax.axis_index("s") * NUM_CORES + lax.axis_index("c")
    base_row = wid * B_PER_W

    def img_fire(j, _):
        pltpu.async_copy(
            img_hbm.at[pl.ds(j * BATCH + base_row, B_PER_W)],
            img_v.at[pl.ds(j * B_PER_W, B_PER_W)], sem2)
        return _

    def img_wait(j, _):
        pltpu.make_async_copy(
            img_hbm.at[pl.ds(j * BATCH + base_row, B_PER_W)],
            img_v.at[pl.ds(j * B_PER_W, B_PER_W)], sem2).wait()
        return _

    lax.fori_loop(0, N_BITS, img_fire, 0)
    lax.fori_loop(0, N_BITS, img_wait, 0)
    pl.run_scoped(lambda gsem: _blocks(enc_hbm, out_hbm, scale_v, addr_v,
                                       gat_v, img_v, sem2, gsem, wid),
                  pltpu.SemaphoreType.DMA((NBLK,)))


def _blocks(enc_hbm, out_hbm, scale_v, addr_v, gat_v, img_v, sem2, gsem, wid):

    def block_front(b2, _):
        def index_chunk(c8, _):
            cix = b2 * 8 + c8
            idx = jnp.zeros((LANES,), jnp.int32)
            acc = jnp.zeros((LANES,), jnp.float32)
            for j in range(N_BITS):
                g = img_v[pl.ds(j * B_PER_W + cix * LANES, LANES)]
                bit = jnp.full((LANES,), 1 << j, jnp.int32)
                idx = idx + jnp.where(g > 0, bit,
                                      jnp.zeros((LANES,), jnp.int32))
                acc = acc + jnp.abs(g)
            scale_v[pl.ds(cix * LANES, LANES)] = acc * (1.0 / N_BITS)
            ebase = ((idx >> 7) << 10) + (idx & 127)
            for h in range(2):
                for cc in range(8):
                    a = ebase + (h * HALF + cc * 128)
                    dst = (((h * NBLK + b2) * 8 + cc) * 8 + c8) * LANES
                    addr_v[pl.ds(dst, LANES)] = a
            return _

        def gat_fire(cc, _):
            for h in range(2):
                o = ((h * NBLK + b2) * 8 + cc) * 128
                pltpu.async_copy(enc_hbm.at[addr_v.at[pl.ds(o, 128)]],
                                 gat_v.at[pl.ds(o, 128)], gsem.at[b2])
            return _

        lax.fori_loop(0, 8, index_chunk, 0)
        lax.fori_loop(0, 8, gat_fire, 0)
        return _

    lax.fori_loop(0, NBLK, block_front, 0)

    half_w = NBLK * 8 * 128

    def block_back(b2, _):
        def gat_wait(cc, _):
            for h in range(2):
                o = ((h * NBLK + b2) * 8 + cc) * 128
                pltpu.make_async_copy(enc_hbm.at[addr_v.at[pl.ds(o, 128)]],
                                      gat_v.at[pl.ds(o, 128)],
                                      gsem.at[b2]).wait()
            return _

        def scale_r8(r8, _):
            s = scale_v[pl.ds(b2 * 128 + r8 * LANES, LANES)]
            for h in range(2):
                for cc in range(8):
                    o = ((h * NBLK + b2) * 8 + cc) * 128 + r8 * LANES
                    gat_v[pl.ds(o, LANES)] = gat_v[pl.ds(o, LANES)] * s
            return _

        lax.fori_loop(0, 8, gat_wait, 0)
        lax.fori_loop(0, 8, scale_r8, 0)
        for h in range(2):
            src_o = (h * NBLK + b2) * 1024
            dst_o = h * (BATCH // 128) * 1024 + wid * half_w + b2 * 1024
            pltpu.async_copy(gat_v.at[pl.ds(src_o, 1024)],
                             out_hbm.at[pl.ds(dst_o, 1024)], sem2)
        return _

    lax.fori_loop(0, NBLK, block_back, 0)

    def out_wait(b2, _):
        for h in range(2):
            src_o = (h * NBLK + b2) * 1024
            dst_o = h * (BATCH // 128) * 1024 + wid * half_w + b2 * 1024
            pltpu.make_async_copy(gat_v.at[pl.ds(src_o, 1024)],
                                  out_hbm.at[pl.ds(dst_o, 1024)], sem2).wait()
        return _

    lax.fori_loop(0, NBLK, out_wait, 0)


@jax.jit
def kernel(image, encoding):
    mesh = plsc.VectorSubcoreMesh(
        core_axis_name="c", subcore_axis_name="s",
        num_cores=NUM_CORES, num_subcores=NUM_SUBCORES)
    k = functools.partial(
        pl.kernel,
        out_type=jax.ShapeDtypeStruct((2 * (BATCH // 128) * 1024,), jnp.float32),
        mesh=mesh,
        scratch_types=[
            pltpu.VMEM((B_PER_W * N_BITS,), jnp.float32),
            pltpu.VMEM((B_PER_W,), jnp.float32),
            pltpu.VMEM((B_PER_W * OUT_DIM,), jnp.int32),
            pltpu.VMEM((B_PER_W * OUT_DIM,), jnp.float32),
            pltpu.SemaphoreType.DMA,
        ],
        compiler_params=pltpu.CompilerParams(
            needs_layout_passes=False, use_tc_tiling_on_sc=False),
    )(_body)
    enc_flat = (encoding.reshape(8192, 128, 2, 8)
                .transpose(2, 0, 3, 1).reshape(-1))
    out = k(image.T.reshape(-1), enc_flat)
    return (out.reshape(2, BATCH // 128, 8, 128)
            .transpose(1, 3, 0, 2).reshape(BATCH, OUT_DIM))
